# baseline (speedup 1.0000x reference)
import dataclasses
import functools

import jax
import jax.numpy as jnp
from jax import lax
from jax.experimental import pallas as pl
from jax.experimental.pallas import tpu as pltpu
from jax.experimental.pallas import tpu_sc as plsc

NC = 2
NS = 16
NW = NC * NS
LANES = 16

_SC_PARAMS = pltpu.CompilerParams()
if "needs_layout_passes" in pltpu.CompilerParams.__dataclass_fields__:
    _SC_PARAMS = dataclasses.replace(_SC_PARAMS, needs_layout_passes=False)


def _tables_body(x_ref, w_ref, src_ref, dst_ref):
    x = x_ref[...].astype(jnp.bfloat16)
    y = jnp.dot(x, w_ref[...], preferred_element_type=jnp.float32)
    src_ref[...] = y[:, :512]
    dst_ref[...] = y[:, 512:]


def _make_tables(atom, w_all):
    n = atom.shape[0]
    bm = 1000
    return pl.pallas_call(
        _tables_body,
        grid=(n // bm,),
        in_specs=[
            pl.BlockSpec((bm, atom.shape[1]), lambda i: (i, 0)),
            pl.BlockSpec(w_all.shape, lambda i: (0, 0)),
        ],
        out_specs=[
            pl.BlockSpec((bm, 512), lambda i: (i, 0)),
            pl.BlockSpec((bm, 512), lambda i: (i, 0)),
        ],
        out_shape=[
            jax.ShapeDtypeStruct((n, 512), jnp.float32),
            jax.ShapeDtypeStruct((n, 512), jnp.float32),
        ],
    )(atom, w_all)


def _gather_sum(src_tab, dst_tab, src_idx, dst_idx):
    e = src_idx.shape[0]
    ew = e // NW
    ca = 40
    nch = ew // ca
    mesh = plsc.VectorSubcoreMesh(core_axis_name="c", subcore_axis_name="s")

    @functools.partial(
        pl.kernel,
        mesh=mesh,
        out_type=jax.ShapeDtypeStruct((e, 512), jnp.float32),
        scratch_types=[
            pltpu.VMEM((ew,), jnp.int32),
            pltpu.VMEM((ew,), jnp.int32),
            pltpu.VMEM((ca, 512), jnp.float32),
            pltpu.VMEM((ca, 512), jnp.float32),
            pltpu.SemaphoreType.DMA,
            pltpu.SemaphoreType.DMA,
        ],
        compiler_params=_SC_PARAMS,
    )
    def pa(src_tab_hbm, dst_tab_hbm, sidx_hbm, didx_hbm, t_hbm,
           sidx_v, didx_v, sbuf, dbuf, sem1, sem2):
        wid = lax.axis_index("s") * NC + lax.axis_index("c")
        base = wid * ew
        pltpu.sync_copy(sidx_hbm.at[pl.ds(base, ew)], sidx_v)
        pltpu.sync_copy(didx_hbm.at[pl.ds(base, ew)], didx_v)

        @pl.loop(0, nch)
        def _chunk(ch):
            off = ch * ca
            cp1 = pltpu.async_copy(
                src_tab_hbm.at[sidx_v.at[pl.ds(off, ca)]], sbuf, sem1)
            cp2 = pltpu.async_copy(
                dst_tab_hbm.at[didx_v.at[pl.ds(off, ca)]], dbuf, sem2)
            cp1.wait()
            cp2.wait()

            @pl.loop(0, ca)
            def _row(r):
                for j in range(512 // LANES):
                    sl = (r, pl.ds(j * LANES, LANES))
                    plsc.addupdate(sbuf.at[sl], dbuf[sl])

            pltpu.sync_copy(sbuf, t_hbm.at[pl.ds(base + off, ca)])

    return pa(src_tab, dst_tab, src_idx, dst_idx)


def _edge_body(t_ref, e_ref, we_ref, b_ref, o_ref):
    q = jnp.dot(e_ref[...], we_ref[...], preferred_element_type=jnp.float32)
    z = t_ref[...] + q + b_ref[0:1, :]
    ts = z[:, :256]
    tg = z[:, 256:]
    sig = 1.0 / (1.0 + jnp.exp(-ts))
    sp = jnp.maximum(tg, 0.0) + jnp.log1p(jnp.exp(-jnp.abs(tg)))
    o_ref[...] = sig * sp


def _edge_mlp(t, edges_bf, we_bf, bias2d):
    e = t.shape[0]
    be = 640
    return pl.pallas_call(
        _edge_body,
        grid=(e // be,),
        in_specs=[
            pl.BlockSpec((be, 512), lambda i: (i, 0)),
            pl.BlockSpec((be, edges_bf.shape[1]), lambda i: (i, 0)),
            pl.BlockSpec(we_bf.shape, lambda i: (0, 0)),
            pl.BlockSpec(bias2d.shape, lambda i: (0, 0)),
        ],
        out_specs=pl.BlockSpec((be, 256), lambda i: (i, 0)),
        out_shape=jax.ShapeDtypeStruct((e, 256), jnp.float32),
    )(t, edges_bf, we_bf, bias2d)


ROWS_PER_TILE = 320
PC_CHUNK = 64


def _segment_add(atom_pad, t, src_idx, bounds):
    e = t.shape[0]
    npad = atom_pad.shape[0]
    cc = PC_CHUNK
    rpt = ROWS_PER_TILE
    mesh = plsc.VectorSubcoreMesh(core_axis_name="c", subcore_axis_name="s")

    @functools.partial(
        pl.kernel,
        mesh=mesh,
        out_type=jax.ShapeDtypeStruct((npad, 256), jnp.float32),
        scratch_types=[
            pltpu.VMEM((rpt, 256), jnp.float32),
            pltpu.VMEM((cc, 256), jnp.float32),
            pltpu.VMEM((cc,), jnp.int32),
            pltpu.VMEM((48,), jnp.int32),
        ],
        compiler_params=_SC_PARAMS,
    )
    def pc(atom_hbm, t_hbm, sidx_hbm, bnd_hbm, out_hbm, slab, tbuf, ibuf,
           bnd_v):
        w = lax.axis_index("s") * NC + lax.axis_index("c")
        nbase = w * rpt
        pltpu.sync_copy(atom_hbm.at[pl.ds(nbase, rpt)], slab)
        pltpu.sync_copy(bnd_hbm, bnd_v)

        iota = lax.iota(jnp.int32, LANES)
        wp = w + 1
        va = bnd_v[pl.ds((w // LANES) * LANES, LANES)]
        e_lo = jnp.sum(jnp.where(iota == w % LANES, va, 0))
        vb = bnd_v[pl.ds((wp // LANES) * LANES, LANES)]
        e_hi = jnp.sum(jnp.where(iota == wp % LANES, vb, 0))
        astart = (e_lo // 8) * 8
        nch = (e_hi - astart + cc - 1) // cc

        def chunk(ch, _):
            eoff = astart + ch * cc
            ero = jnp.minimum(eoff, e - cc)
            cur_lo = jnp.maximum(e_lo, eoff)
            pltpu.sync_copy(t_hbm.at[pl.ds(ero, cc)], tbuf)
            pltpu.sync_copy(sidx_hbm.at[pl.ds(ero, cc)], ibuf)
            for g in range(cc // LANES):
                v = ibuf[pl.ds(g * LANES, LANES)]
                lv = jnp.clip(v - nbase, 0, rpt - 1)
                pos = ero + g * LANES + iota
                msk = (pos >= cur_lo) & (pos < e_hi)
                evec = g * LANES + iota

                @pl.loop(0, 256 // 8)
                def _cols(cg):
                    c0 = jnp.broadcast_to(cg * 8, (LANES,)).astype(jnp.int32)
                    for k in range(8):
                        cv = c0 + k
                        vals = plsc.load_gather(tbuf, [evec, cv])
                        plsc.addupdate_scatter(slab, [lv, cv], vals, mask=msk)

            return 0

        lax.fori_loop(0, nch, chunk, 0)
        pltpu.sync_copy(slab, out_hbm.at[pl.ds(nbase, rpt)])

    return pc(atom_pad, t, src_idx, bounds)


def _softplus_body(x_ref, o_ref):
    x = x_ref[...]
    o_ref[...] = jnp.maximum(x, 0.0) + jnp.log1p(jnp.exp(-jnp.abs(x)))


def _softplus(x):
    n = x.shape[0]
    bm = 2048
    return pl.pallas_call(
        _softplus_body,
        grid=(n // bm,),
        in_specs=[pl.BlockSpec((bm, x.shape[1]), lambda i: (i, 0))],
        out_specs=pl.BlockSpec((bm, x.shape[1]), lambda i: (i, 0)),
        out_shape=jax.ShapeDtypeStruct(x.shape, jnp.float32),
    )(x)


def kernel(atom_features, edges_features, pair_indices, kernel_s, bias_s,
           kernel_g, bias_g):
    d = atom_features.shape[1]
    src = pair_indices[:, 0]
    dst = pair_indices[:, 1]

    w_all = jnp.concatenate(
        [kernel_s[:d], kernel_g[:d], kernel_s[d:2 * d], kernel_g[d:2 * d]],
        axis=1).astype(jnp.bfloat16)
    we = jnp.concatenate([kernel_s[2 * d:], kernel_g[2 * d:]],
                         axis=1).astype(jnp.bfloat16)
    bias2d = jnp.tile(jnp.concatenate([bias_s, bias_g])[None, :], (8, 1))

    n = atom_features.shape[0]
    npad = NW * ROWS_PER_TILE
    atom_pad = jnp.pad(atom_features, ((0, npad - n), (0, 0)))
    bounds = jnp.pad(
        jnp.searchsorted(
            src, jnp.arange(NW + 1, dtype=jnp.int32) * ROWS_PER_TILE,
            side="left").astype(jnp.int32),
        (0, 48 - (NW + 1)), constant_values=src.shape[0])

    src_tab, dst_tab = _make_tables(atom_features, w_all)
    t = _gather_sum(src_tab, dst_tab, src, dst)
    tact = _edge_mlp(t, edges_features.astype(jnp.bfloat16), we, bias2d)
    agg = _segment_add(atom_pad, tact, src, bounds)
    return _softplus(agg)[:n]


# speedup vs baseline: 2.2349x; 2.2349x over previous
import dataclasses
import functools

import jax
import jax.numpy as jnp
from jax import lax
from jax.experimental import pallas as pl
from jax.experimental.pallas import tpu as pltpu
from jax.experimental.pallas import tpu_sc as plsc

NC = 2
NS = 16
NW = NC * NS
LANES = 16

_SC_PARAMS = pltpu.CompilerParams()
if "needs_layout_passes" in pltpu.CompilerParams.__dataclass_fields__:
    _SC_PARAMS = dataclasses.replace(_SC_PARAMS, needs_layout_passes=False)


def _tables_body(x_ref, w_ref, src_ref, dst_ref):
    x = x_ref[...].astype(jnp.bfloat16)
    y = jnp.dot(x, w_ref[...], preferred_element_type=jnp.float32)
    src_ref[...] = y[:, :512]
    dst_ref[...] = y[:, 512:]


def _make_tables(atom, w_all):
    n = atom.shape[0]
    bm = 1000
    return pl.pallas_call(
        _tables_body,
        grid=(n // bm,),
        in_specs=[
            pl.BlockSpec((bm, atom.shape[1]), lambda i: (i, 0)),
            pl.BlockSpec(w_all.shape, lambda i: (0, 0)),
        ],
        out_specs=[
            pl.BlockSpec((bm, 512), lambda i: (i, 0)),
            pl.BlockSpec((bm, 512), lambda i: (i, 0)),
        ],
        out_shape=[
            jax.ShapeDtypeStruct((n, 512), jnp.float32),
            jax.ShapeDtypeStruct((n, 512), jnp.float32),
        ],
    )(atom, w_all)


def _gather_sum(src_tab, dst_tab, src_idx, dst_idx):
    e = src_idx.shape[0]
    ew = e // NW
    ca = 32
    nfull = ew // ca
    tail = ew - nfull * ca
    npairs = nfull // 2
    odd = nfull - npairs * 2
    mesh = plsc.VectorSubcoreMesh(core_axis_name="c", subcore_axis_name="s")

    @functools.partial(
        pl.kernel,
        mesh=mesh,
        out_type=jax.ShapeDtypeStruct((e, 512), jnp.float32),
        scratch_types=[
            pltpu.VMEM((ew,), jnp.int32),
            pltpu.VMEM((ew,), jnp.int32),
            pltpu.VMEM((ca, 512), jnp.float32),
            pltpu.VMEM((ca, 512), jnp.float32),
            pltpu.VMEM((ca, 512), jnp.float32),
            pltpu.VMEM((ca, 512), jnp.float32),
            pltpu.VMEM((ca, 512), jnp.float32),
            pltpu.VMEM((ca, 512), jnp.float32),
            pltpu.SemaphoreType.DMA,
            pltpu.SemaphoreType.DMA,
            pltpu.SemaphoreType.DMA,
            pltpu.SemaphoreType.DMA,
            pltpu.SemaphoreType.DMA,
            pltpu.SemaphoreType.DMA,
        ],
        compiler_params=_SC_PARAMS,
    )
    def pa(src_tab_hbm, dst_tab_hbm, sidx_hbm, didx_hbm, t_hbm,
           sidx_v, didx_v, s0, d0, s1, d1, o0, o1,
           gs0, gd0, gs1, gd1, st0, st1):
        wid = lax.axis_index("s") * NC + lax.axis_index("c")
        base = wid * ew
        pltpu.sync_copy(sidx_hbm.at[pl.ds(base, ew)], sidx_v)
        pltpu.sync_copy(didx_hbm.at[pl.ds(base, ew)], didx_v)

        def g_issue(off, nr, sb, db, ss, sd):
            pltpu.make_async_copy(
                src_tab_hbm.at[sidx_v.at[pl.ds(off, nr)]], sb, ss).start()
            pltpu.make_async_copy(
                dst_tab_hbm.at[didx_v.at[pl.ds(off, nr)]], db, sd).start()

        def g_wait(off, nr, sb, db, ss, sd):
            pltpu.make_async_copy(
                src_tab_hbm.at[sidx_v.at[pl.ds(off, nr)]], sb, ss).wait()
            pltpu.make_async_copy(
                dst_tab_hbm.at[didx_v.at[pl.ds(off, nr)]], db, sd).wait()

        def do_add(nr, sb, db, ob):
            @pl.loop(0, nr)
            def _row(r):
                for j in range(512 // LANES):
                    sl = (r, pl.ds(j * LANES, LANES))
                    ob[sl] = sb[sl] + db[sl]

        def st_issue(off, nr, ob, sem):
            pltpu.make_async_copy(
                ob, t_hbm.at[pl.ds(base + off, nr)], sem).start()

        def st_wait(off, nr, ob, sem):
            pltpu.make_async_copy(
                ob, t_hbm.at[pl.ds(base + off, nr)], sem).wait()

        g_issue(0, ca, s0, d0, gs0, gd0)

        def pair(p, _):
            off0 = 2 * p * ca
            off1 = off0 + ca
            off2 = off1 + ca
            g_issue(off1, ca, s1, d1, gs1, gd1)
            g_wait(off0, ca, s0, d0, gs0, gd0)

            @pl.when(p > 0)
            def _w0():
                st_wait(off0, ca, o0, st0)

            do_add(ca, s0, d0, o0)
            st_issue(off0, ca, o0, st0)

            @pl.when(off2 < nfull * ca)
            def _nx():
                g_issue(off2, ca, s0, d0, gs0, gd0)

            g_wait(off1, ca, s1, d1, gs1, gd1)

            @pl.when(p > 0)
            def _w1():
                st_wait(off1, ca, o1, st1)

            do_add(ca, s1, d1, o1)
            st_issue(off1, ca, o1, st1)
            return 0

        lax.fori_loop(0, npairs, pair, 0)
        assert odd == 0, "pair loop expects an even number of full chunks"
        st_wait(0, ca, o1, st1)
        if tail > 0:
            toff = nfull * ca
            sbt, dbt = s0.at[pl.ds(0, tail)], d0.at[pl.ds(0, tail)]
            obt = o0.at[pl.ds(0, tail)]
            g_issue(toff, tail, sbt, dbt, gs0, gd0)
            st_wait(0, ca, o0, st0)
            g_wait(toff, tail, sbt, dbt, gs0, gd0)
            do_add(tail, s0, d0, o0)
            pltpu.make_async_copy(
                obt, t_hbm.at[pl.ds(base + toff, tail)], st0).start()
            pltpu.make_async_copy(
                obt, t_hbm.at[pl.ds(base + toff, tail)], st0).wait()
        else:
            st_wait(0, ca, o0, st0)

    return pa(src_tab, dst_tab, src_idx, dst_idx)


def _edge_body(t_ref, e_ref, we_ref, b_ref, o_ref):
    q = jnp.dot(e_ref[...], we_ref[...], preferred_element_type=jnp.float32)
    z = t_ref[...] + q + b_ref[0:1, :]
    ts = z[:, :256]
    tg = z[:, 256:]
    sig = 1.0 / (1.0 + jnp.exp(-ts))
    sp = jnp.maximum(tg, 0.0) + jnp.log1p(jnp.exp(-jnp.abs(tg)))
    o_ref[...] = sig * sp


def _edge_mlp(t, edges_bf, we_bf, bias2d):
    e = t.shape[0]
    be = 640
    return pl.pallas_call(
        _edge_body,
        grid=(e // be,),
        in_specs=[
            pl.BlockSpec((be, 512), lambda i: (i, 0)),
            pl.BlockSpec((be, edges_bf.shape[1]), lambda i: (i, 0)),
            pl.BlockSpec(we_bf.shape, lambda i: (0, 0)),
            pl.BlockSpec(bias2d.shape, lambda i: (0, 0)),
        ],
        out_specs=pl.BlockSpec((be, 256), lambda i: (i, 0)),
        out_shape=jax.ShapeDtypeStruct((e, 256), jnp.float32),
    )(t, edges_bf, we_bf, bias2d)


ROWS_PER_TILE = 320
PC_CHUNK = 64


def _segment_add(atom_pad, t, src_idx, bounds):
    e = t.shape[0]
    npad = atom_pad.shape[0]
    cc = PC_CHUNK
    rpt = ROWS_PER_TILE
    mesh = plsc.VectorSubcoreMesh(core_axis_name="c", subcore_axis_name="s")

    @functools.partial(
        pl.kernel,
        mesh=mesh,
        out_type=jax.ShapeDtypeStruct((npad * 256,), jnp.float32),
        scratch_types=[
            pltpu.VMEM((rpt * 256,), jnp.float32),
            pltpu.VMEM((cc * 256,), jnp.float32),
            pltpu.VMEM((cc,), jnp.int32),
            pltpu.VMEM((48,), jnp.int32),
        ],
        compiler_params=_SC_PARAMS,
    )
    def pc(atom_hbm, t_hbm, sidx_hbm, bnd_hbm, out_hbm, slab, tbuf, ibuf,
           bnd_v):
        w = lax.axis_index("s") * NC + lax.axis_index("c")
        nbase = w * rpt
        pltpu.sync_copy(atom_hbm.at[pl.ds(nbase * 256, rpt * 256)], slab)
        pltpu.sync_copy(bnd_hbm, bnd_v)

        iota = lax.iota(jnp.int32, LANES)
        wp = w + 1
        va = bnd_v[pl.ds((w // LANES) * LANES, LANES)]
        e_lo = jnp.sum(jnp.where(iota == w % LANES, va, 0))
        vb = bnd_v[pl.ds((wp // LANES) * LANES, LANES)]
        e_hi = jnp.sum(jnp.where(iota == wp % LANES, vb, 0))
        astart = (e_lo // 8) * 8
        nch = (e_hi - astart + cc - 1) // cc

        coloffs = [jnp.bitwise_and(iota + k, LANES - 1) for k in range(LANES)]
        ebase = [(g * LANES + iota) * 256 for g in range(cc // LANES)]

        def chunk(ch, _):
            eoff = astart + ch * cc
            ero = jnp.minimum(eoff, e - cc)
            cur_lo = jnp.maximum(e_lo, eoff)
            pltpu.sync_copy(t_hbm.at[pl.ds(ero * 256, cc * 256)], tbuf)
            pltpu.sync_copy(sidx_hbm.at[pl.ds(ero, cc)], ibuf)
            for g in range(cc // LANES):
                v = ibuf[pl.ds(g * LANES, LANES)]
                rowb = jnp.clip(v - nbase, 0, rpt - 1) * 256
                pos = ero + g * LANES + iota
                msk = (pos >= cur_lo) & (pos < e_hi)
                eb = ebase[g]

                @pl.loop(0, 256 // LANES)
                def _cols(cg):
                    c0 = cg * LANES
                    sb = rowb + c0
                    tb = eb + c0
                    for k in range(LANES):
                        co = coloffs[k]
                        vals = plsc.load_gather(tbuf, [tb + co])
                        plsc.addupdate_scatter(slab, [sb + co], vals,
                                               mask=msk)

            return 0

        lax.fori_loop(0, nch, chunk, 0)
        pltpu.sync_copy(slab, out_hbm.at[pl.ds(nbase * 256, rpt * 256)])

    out = pc(atom_pad.reshape(-1), t.reshape(-1), src_idx, bounds)
    return out.reshape(npad, 256)


def _softplus_body(x_ref, o_ref):
    x = x_ref[...]
    o_ref[...] = jnp.maximum(x, 0.0) + jnp.log1p(jnp.exp(-jnp.abs(x)))


def _softplus(x):
    n = x.shape[0]
    bm = 2048
    return pl.pallas_call(
        _softplus_body,
        grid=(n // bm,),
        in_specs=[pl.BlockSpec((bm, x.shape[1]), lambda i: (i, 0))],
        out_specs=pl.BlockSpec((bm, x.shape[1]), lambda i: (i, 0)),
        out_shape=jax.ShapeDtypeStruct(x.shape, jnp.float32),
    )(x)


def kernel(atom_features, edges_features, pair_indices, kernel_s, bias_s,
           kernel_g, bias_g):
    d = atom_features.shape[1]
    src = pair_indices[:, 0]
    dst = pair_indices[:, 1]

    w_all = jnp.concatenate(
        [kernel_s[:d], kernel_g[:d], kernel_s[d:2 * d], kernel_g[d:2 * d]],
        axis=1).astype(jnp.bfloat16)
    we = jnp.concatenate([kernel_s[2 * d:], kernel_g[2 * d:]],
                         axis=1).astype(jnp.bfloat16)
    bias2d = jnp.tile(jnp.concatenate([bias_s, bias_g])[None, :], (8, 1))

    n = atom_features.shape[0]
    npad = NW * ROWS_PER_TILE
    atom_pad = jnp.pad(atom_features, ((0, npad - n), (0, 0)))
    bounds = jnp.pad(
        jnp.searchsorted(
            src, jnp.arange(NW + 1, dtype=jnp.int32) * ROWS_PER_TILE,
            side="left").astype(jnp.int32),
        (0, 48 - (NW + 1)), constant_values=src.shape[0])

    src_tab, dst_tab = _make_tables(atom_features, w_all)
    t = _gather_sum(src_tab, dst_tab, src, dst)
    tact = _edge_mlp(t, edges_features.astype(jnp.bfloat16), we, bias2d)
    agg = _segment_add(atom_pad, tact, src, bounds)
    return _softplus(agg)[:n]


# speedup vs baseline: 2.4892x; 1.1138x over previous
import dataclasses
import functools

import jax
import jax.numpy as jnp
from jax import lax
from jax.experimental import pallas as pl
from jax.experimental.pallas import tpu as pltpu
from jax.experimental.pallas import tpu_sc as plsc

NC = 2
NS = 16
NW = NC * NS
LANES = 16

_SC_PARAMS = pltpu.CompilerParams()
if "needs_layout_passes" in pltpu.CompilerParams.__dataclass_fields__:
    _SC_PARAMS = dataclasses.replace(_SC_PARAMS, needs_layout_passes=False)


def _tables_body(x_ref, w_ref, src_ref, dst_ref):
    x = x_ref[...].astype(jnp.bfloat16)
    y = jnp.dot(x, w_ref[...], preferred_element_type=jnp.float32)
    src_ref[...] = y[:, :512].astype(jnp.bfloat16)
    dst_ref[...] = y[:, 512:].astype(jnp.bfloat16)


def _make_tables(atom, w_all):
    n = atom.shape[0]
    bm = 1000
    return pl.pallas_call(
        _tables_body,
        grid=(n // bm,),
        in_specs=[
            pl.BlockSpec((bm, atom.shape[1]), lambda i: (i, 0)),
            pl.BlockSpec(w_all.shape, lambda i: (0, 0)),
        ],
        out_specs=[
            pl.BlockSpec((bm, 512), lambda i: (i, 0)),
            pl.BlockSpec((bm, 512), lambda i: (i, 0)),
        ],
        out_shape=[
            jax.ShapeDtypeStruct((n, 512), jnp.bfloat16),
            jax.ShapeDtypeStruct((n, 512), jnp.bfloat16),
        ],
    )(atom, w_all)


def _gather_sum(src_tab, dst_tab, src_idx, dst_idx):
    e = src_idx.shape[0]
    ew = e // NW
    ca = 64
    nfull = ew // ca
    tail = ew - nfull * ca
    npairs = nfull // 2
    odd = nfull - npairs * 2
    mesh = plsc.VectorSubcoreMesh(core_axis_name="c", subcore_axis_name="s")

    @functools.partial(
        pl.kernel,
        mesh=mesh,
        out_type=jax.ShapeDtypeStruct((e, 256), jnp.int32),
        scratch_types=[
            pltpu.VMEM((ew,), jnp.int32),
            pltpu.VMEM((ew,), jnp.int32),
            pltpu.VMEM((ca, 256), jnp.int32),
            pltpu.VMEM((ca, 256), jnp.int32),
            pltpu.VMEM((ca, 256), jnp.int32),
            pltpu.VMEM((ca, 256), jnp.int32),
            pltpu.VMEM((ca, 256), jnp.int32),
            pltpu.VMEM((ca, 256), jnp.int32),
            pltpu.SemaphoreType.DMA,
            pltpu.SemaphoreType.DMA,
            pltpu.SemaphoreType.DMA,
            pltpu.SemaphoreType.DMA,
            pltpu.SemaphoreType.DMA,
            pltpu.SemaphoreType.DMA,
        ],
        compiler_params=_SC_PARAMS,
    )
    def pa(src_tab_hbm, dst_tab_hbm, sidx_hbm, didx_hbm, t_hbm,
           sidx_v, didx_v, s0, d0, s1, d1, o0, o1,
           gs0, gd0, gs1, gd1, st0, st1):
        wid = lax.axis_index("s") * NC + lax.axis_index("c")
        base = wid * ew
        pltpu.sync_copy(sidx_hbm.at[pl.ds(base, ew)], sidx_v)
        pltpu.sync_copy(didx_hbm.at[pl.ds(base, ew)], didx_v)

        def g_issue(off, nr, sb, db, ss, sd):
            pltpu.make_async_copy(
                src_tab_hbm.at[sidx_v.at[pl.ds(off, nr)]], sb, ss).start()
            pltpu.make_async_copy(
                dst_tab_hbm.at[didx_v.at[pl.ds(off, nr)]], db, sd).start()

        def g_wait(off, nr, sb, db, ss, sd):
            pltpu.make_async_copy(
                src_tab_hbm.at[sidx_v.at[pl.ds(off, nr)]], sb, ss).wait()
            pltpu.make_async_copy(
                dst_tab_hbm.at[didx_v.at[pl.ds(off, nr)]], db, sd).wait()

        def do_add(nr, sb, db, ob):
            @pl.loop(0, nr)
            def _row(r):
                for j in range(256 // LANES):
                    sl = (r, pl.ds(j * LANES, LANES))
                    a = plsc.bitcast(sb[sl], jnp.bfloat16)
                    b = plsc.bitcast(db[sl], jnp.bfloat16)
                    ob[sl] = plsc.bitcast(a + b, jnp.int32)

        def st_issue(off, nr, ob, sem):
            pltpu.make_async_copy(
                ob, t_hbm.at[pl.ds(base + off, nr)], sem).start()

        def st_wait(off, nr, ob, sem):
            pltpu.make_async_copy(
                ob, t_hbm.at[pl.ds(base + off, nr)], sem).wait()

        g_issue(0, ca, s0, d0, gs0, gd0)

        def pair(p, _):
            off0 = 2 * p * ca
            off1 = off0 + ca
            off2 = off1 + ca
            g_issue(off1, ca, s1, d1, gs1, gd1)
            g_wait(off0, ca, s0, d0, gs0, gd0)

            @pl.when(p > 0)
            def _w0():
                st_wait(off0, ca, o0, st0)

            do_add(ca, s0, d0, o0)
            st_issue(off0, ca, o0, st0)

            @pl.when(off2 < nfull * ca)
            def _nx():
                g_issue(off2, ca, s0, d0, gs0, gd0)

            g_wait(off1, ca, s1, d1, gs1, gd1)

            @pl.when(p > 0)
            def _w1():
                st_wait(off1, ca, o1, st1)

            do_add(ca, s1, d1, o1)
            st_issue(off1, ca, o1, st1)
            return 0

        lax.fori_loop(0, npairs, pair, 0)
        assert odd == 0, "pair loop expects an even number of full chunks"
        st_wait(0, ca, o1, st1)
        if tail > 0:
            toff = nfull * ca
            sbt, dbt = s0.at[pl.ds(0, tail)], d0.at[pl.ds(0, tail)]
            obt = o0.at[pl.ds(0, tail)]
            g_issue(toff, tail, sbt, dbt, gs0, gd0)
            st_wait(0, ca, o0, st0)
            g_wait(toff, tail, sbt, dbt, gs0, gd0)
            do_add(tail, s0, d0, o0)
            pltpu.make_async_copy(
                obt, t_hbm.at[pl.ds(base + toff, tail)], st0).start()
            pltpu.make_async_copy(
                obt, t_hbm.at[pl.ds(base + toff, tail)], st0).wait()
        else:
            st_wait(0, ca, o0, st0)

    return pa(src_tab, dst_tab, src_idx, dst_idx)


def _edge_body(t_ref, e_ref, we_ref, b_ref, o_ref):
    w32 = t_ref[...]
    q = jnp.dot(e_ref[...], we_ref[...], preferred_element_type=jnp.float32)
    tsv = jax.lax.bitcast_convert_type(
        jax.lax.shift_left(w32, 16), jnp.float32)
    tgv = jax.lax.bitcast_convert_type(
        jax.lax.bitwise_and(w32, jnp.int32(-65536)), jnp.float32)
    ts = tsv + q[:, :256] + b_ref[0:1, :256]
    tg = tgv + q[:, 256:] + b_ref[0:1, 256:]
    sig = 1.0 / (1.0 + jnp.exp(-ts))
    sp = jnp.maximum(tg, 0.0) + jnp.log1p(jnp.exp(-jnp.abs(tg)))
    o_ref[...] = sig * sp


def _edge_mlp(t, edges_bf, we_bf, bias2d):
    e = t.shape[0]
    be = 640
    return pl.pallas_call(
        _edge_body,
        grid=(e // be,),
        in_specs=[
            pl.BlockSpec((be, 256), lambda i: (i, 0)),
            pl.BlockSpec((be, edges_bf.shape[1]), lambda i: (i, 0)),
            pl.BlockSpec(we_bf.shape, lambda i: (0, 0)),
            pl.BlockSpec(bias2d.shape, lambda i: (0, 0)),
        ],
        out_specs=pl.BlockSpec((be, 256), lambda i: (i, 0)),
        out_shape=jax.ShapeDtypeStruct((e, 256), jnp.float32),
    )(t, edges_bf, we_bf, bias2d)


ROWS_PER_TILE = 320
PC_CHUNK = 64


def _segment_add(atom_pad, t, src_idx, bounds):
    e = t.shape[0]
    npad = atom_pad.shape[0]
    cc = PC_CHUNK
    rpt = ROWS_PER_TILE
    mesh = plsc.VectorSubcoreMesh(core_axis_name="c", subcore_axis_name="s")

    @functools.partial(
        pl.kernel,
        mesh=mesh,
        out_type=jax.ShapeDtypeStruct((npad, 256), jnp.float32),
        scratch_types=[
            pltpu.VMEM((rpt, 256), jnp.float32),
            pltpu.VMEM((cc, 256), jnp.float32),
            pltpu.VMEM((cc, 256), jnp.float32),
            pltpu.VMEM((cc,), jnp.int32),
            pltpu.VMEM((cc,), jnp.int32),
            pltpu.VMEM((48,), jnp.int32),
            pltpu.SemaphoreType.DMA,
            pltpu.SemaphoreType.DMA,
        ],
        compiler_params=_SC_PARAMS,
    )
    def pc(atom_hbm, t_hbm, sidx_hbm, bnd_hbm, out_hbm, slab, tb0, tb1,
           ib0, ib1, bnd_v, sm0, sm1):
        w = lax.axis_index("s") * NC + lax.axis_index("c")
        nbase = w * rpt
        pltpu.sync_copy(atom_hbm.at[pl.ds(nbase, rpt)], slab)
        pltpu.sync_copy(bnd_hbm, bnd_v)

        iota = lax.iota(jnp.int32, LANES)
        wp = w + 1
        va = bnd_v[pl.ds((w // LANES) * LANES, LANES)]
        e_lo = jnp.sum(jnp.where(iota == w % LANES, va, 0))
        vb = bnd_v[pl.ds((wp // LANES) * LANES, LANES)]
        e_hi = jnp.sum(jnp.where(iota == wp % LANES, vb, 0))
        astart = (e_lo // 8) * 8
        nch = (e_hi - astart + cc - 1) // cc

        coloffs = [jnp.bitwise_and(iota + k, LANES - 1) for k in range(LANES)]
        erow = [g * LANES + iota for g in range(cc // LANES)]

        def ch_ero(ch):
            return jnp.minimum(astart + ch * cc, e - cc)

        def issue(ch, tb, ib, sm):
            ero = ch_ero(ch)
            pltpu.make_async_copy(t_hbm.at[pl.ds(ero, cc)], tb, sm).start()
            pltpu.make_async_copy(sidx_hbm.at[pl.ds(ero, cc)], ib, sm).start()

        def wait(ch, tb, ib, sm):
            ero = ch_ero(ch)
            pltpu.make_async_copy(t_hbm.at[pl.ds(ero, cc)], tb, sm).wait()
            pltpu.make_async_copy(sidx_hbm.at[pl.ds(ero, cc)], ib, sm).wait()

        def process(ch, tb, ib):
            eoff = astart + ch * cc
            ero = ch_ero(ch)
            cur_lo = jnp.maximum(e_lo, eoff)
            for g in range(cc // LANES):
                v = ib[pl.ds(g * LANES, LANES)]
                lv = jnp.clip(v - nbase, 0, rpt - 1)
                pos = ero + g * LANES + iota
                msk = (pos >= cur_lo) & (pos < e_hi)
                ev = erow[g]

                @pl.loop(0, 256 // LANES)
                def _cols(cg):
                    c0 = cg * LANES
                    for k in range(LANES):
                        cv = c0 + coloffs[k]
                        vals = plsc.load_gather(tb, [ev, cv])
                        plsc.addupdate_scatter(slab, [lv, cv], vals,
                                               mask=msk)

        @pl.when(nch > 0)
        def _pro():
            issue(0, tb0, ib0, sm0)

        def pair(p, _):
            ch0 = 2 * p
            ch1 = ch0 + 1

            @pl.when(ch1 < nch)
            def _i1():
                issue(ch1, tb1, ib1, sm1)

            wait(ch0, tb0, ib0, sm0)
            process(ch0, tb0, ib0)

            @pl.when(ch1 < nch)
            def _s1():
                @pl.when(ch1 + 1 < nch)
                def _i2():
                    issue(ch1 + 1, tb0, ib0, sm0)

                wait(ch1, tb1, ib1, sm1)
                process(ch1, tb1, ib1)

            return 0

        lax.fori_loop(0, (nch + 1) // 2, pair, 0)
        pltpu.sync_copy(slab, out_hbm.at[pl.ds(nbase, rpt)])

    return pc(atom_pad, t, src_idx, bounds)


def _softplus_body(x_ref, o_ref):
    x = x_ref[...]
    o_ref[...] = jnp.maximum(x, 0.0) + jnp.log1p(jnp.exp(-jnp.abs(x)))


def _softplus(x):
    n = x.shape[0]
    bm = 2048
    return pl.pallas_call(
        _softplus_body,
        grid=(n // bm,),
        in_specs=[pl.BlockSpec((bm, x.shape[1]), lambda i: (i, 0))],
        out_specs=pl.BlockSpec((bm, x.shape[1]), lambda i: (i, 0)),
        out_shape=jax.ShapeDtypeStruct(x.shape, jnp.float32),
    )(x)


def kernel(atom_features, edges_features, pair_indices, kernel_s, bias_s,
           kernel_g, bias_g):
    d = atom_features.shape[1]
    src = pair_indices[:, 0]
    dst = pair_indices[:, 1]

    src_inter = jnp.stack([kernel_s[:d], kernel_g[:d]], axis=2).reshape(d, 512)
    dst_inter = jnp.stack([kernel_s[d:2 * d], kernel_g[d:2 * d]],
                          axis=2).reshape(d, 512)
    w_all = jnp.concatenate([src_inter, dst_inter], axis=1).astype(jnp.bfloat16)
    we = jnp.concatenate([kernel_s[2 * d:], kernel_g[2 * d:]],
                         axis=1).astype(jnp.bfloat16)
    bias2d = jnp.tile(jnp.concatenate([bias_s, bias_g])[None, :], (8, 1))

    n = atom_features.shape[0]
    npad = NW * ROWS_PER_TILE
    atom_pad = jnp.pad(atom_features, ((0, npad - n), (0, 0)))
    bounds = jnp.pad(
        jnp.searchsorted(
            src, jnp.arange(NW + 1, dtype=jnp.int32) * ROWS_PER_TILE,
            side="left").astype(jnp.int32),
        (0, 48 - (NW + 1)), constant_values=src.shape[0])

    src_tab, dst_tab = _make_tables(atom_features, w_all)
    src_tab32 = jax.lax.bitcast_convert_type(
        src_tab.reshape(n, 256, 2), jnp.int32)
    dst_tab32 = jax.lax.bitcast_convert_type(
        dst_tab.reshape(n, 256, 2), jnp.int32)
    t = _gather_sum(src_tab32, dst_tab32, src, dst)
    tact = _edge_mlp(t, edges_features.astype(jnp.bfloat16), we, bias2d)
    agg = _segment_add(atom_pad, tact, src, bounds)
    return _softplus(agg)[:n]


# speedup vs baseline: 3.4500x; 1.3860x over previous
import dataclasses
import functools

import jax
import jax.numpy as jnp
from jax import lax
from jax.experimental import pallas as pl
from jax.experimental.pallas import tpu as pltpu
from jax.experimental.pallas import tpu_sc as plsc

NC = 2
NS = 16
NW = NC * NS
LANES = 16

_SC_PARAMS = pltpu.CompilerParams()
if "needs_layout_passes" in pltpu.CompilerParams.__dataclass_fields__:
    _SC_PARAMS = dataclasses.replace(_SC_PARAMS, needs_layout_passes=False)


def _pack_bf16_pair(s, g):
    u = jax.lax.bitcast_convert_type(s, jnp.int32)
    v = jax.lax.bitcast_convert_type(g, jnp.int32)
    one = jnp.int32(1)
    u = u + 0x7FFF + jax.lax.bitwise_and(jax.lax.shift_right_logical(u, 16),
                                         one)
    v = v + 0x7FFF + jax.lax.bitwise_and(jax.lax.shift_right_logical(v, 16),
                                         one)
    lo = jax.lax.shift_right_logical(u, 16)
    hi = jax.lax.bitwise_and(v, jnp.int32(-65536))
    return jax.lax.bitwise_or(lo, hi)


def _tables_body(x_ref, w_ref, src_ref, dst_ref):
    x = x_ref[...].astype(jnp.bfloat16)
    y = jnp.dot(x, w_ref[...], preferred_element_type=jnp.float32)
    src_ref[...] = _pack_bf16_pair(y[:, :256], y[:, 256:512])
    dst_ref[...] = _pack_bf16_pair(y[:, 512:768], y[:, 768:])


def _make_tables(atom, w_all):
    n = atom.shape[0]
    bm = 1000
    return pl.pallas_call(
        _tables_body,
        grid=(n // bm,),
        in_specs=[
            pl.BlockSpec((bm, atom.shape[1]), lambda i: (i, 0)),
            pl.BlockSpec(w_all.shape, lambda i: (0, 0)),
        ],
        out_specs=[
            pl.BlockSpec((bm, 256), lambda i: (i, 0)),
            pl.BlockSpec((bm, 256), lambda i: (i, 0)),
        ],
        out_shape=[
            jax.ShapeDtypeStruct((n, 256), jnp.int32),
            jax.ShapeDtypeStruct((n, 256), jnp.int32),
        ],
    )(atom, w_all)


def _gather_sum(src_tab, dst_tab, src_idx, dst_idx):
    e = src_idx.shape[0]
    ew = e // NW
    ca = 64
    nfull = ew // ca
    tail = ew - nfull * ca
    npairs = nfull // 2
    odd = nfull - npairs * 2
    mesh = plsc.VectorSubcoreMesh(core_axis_name="c", subcore_axis_name="s")

    @functools.partial(
        pl.kernel,
        mesh=mesh,
        out_type=jax.ShapeDtypeStruct((e, 256), jnp.int32),
        scratch_types=[
            pltpu.VMEM((ew,), jnp.int32),
            pltpu.VMEM((ew,), jnp.int32),
            pltpu.VMEM((ca, 256), jnp.int32),
            pltpu.VMEM((ca, 256), jnp.int32),
            pltpu.VMEM((ca, 256), jnp.int32),
            pltpu.VMEM((ca, 256), jnp.int32),
            pltpu.VMEM((ca, 256), jnp.int32),
            pltpu.VMEM((ca, 256), jnp.int32),
            pltpu.SemaphoreType.DMA,
            pltpu.SemaphoreType.DMA,
            pltpu.SemaphoreType.DMA,
            pltpu.SemaphoreType.DMA,
            pltpu.SemaphoreType.DMA,
            pltpu.SemaphoreType.DMA,
        ],
        compiler_params=_SC_PARAMS,
    )
    def pa(src_tab_hbm, dst_tab_hbm, sidx_hbm, didx_hbm, t_hbm,
           sidx_v, didx_v, s0, d0, s1, d1, o0, o1,
           gs0, gd0, gs1, gd1, st0, st1):
        wid = lax.axis_index("s") * NC + lax.axis_index("c")
        base = wid * ew
        pltpu.sync_copy(sidx_hbm.at[pl.ds(base, ew)], sidx_v)
        pltpu.sync_copy(didx_hbm.at[pl.ds(base, ew)], didx_v)

        def g_issue(off, nr, sb, db, ss, sd):
            pltpu.make_async_copy(
                src_tab_hbm.at[sidx_v.at[pl.ds(off, nr)]], sb, ss).start()
            pltpu.make_async_copy(
                dst_tab_hbm.at[didx_v.at[pl.ds(off, nr)]], db, sd).start()

        def g_wait(off, nr, sb, db, ss, sd):
            pltpu.make_async_copy(
                src_tab_hbm.at[sidx_v.at[pl.ds(off, nr)]], sb, ss).wait()
            pltpu.make_async_copy(
                dst_tab_hbm.at[didx_v.at[pl.ds(off, nr)]], db, sd).wait()

        def do_add(nr, sb, db, ob):
            @pl.loop(0, nr)
            def _row(r):
                for j in range(256 // LANES):
                    sl = (r, pl.ds(j * LANES, LANES))
                    a = plsc.bitcast(sb[sl], jnp.bfloat16)
                    b = plsc.bitcast(db[sl], jnp.bfloat16)
                    ob[sl] = plsc.bitcast(a + b, jnp.int32)

        def st_issue(off, nr, ob, sem):
            pltpu.make_async_copy(
                ob, t_hbm.at[pl.ds(base + off, nr)], sem).start()

        def st_wait(off, nr, ob, sem):
            pltpu.make_async_copy(
                ob, t_hbm.at[pl.ds(base + off, nr)], sem).wait()

        g_issue(0, ca, s0, d0, gs0, gd0)

        def pair(p, _):
            off0 = 2 * p * ca
            off1 = off0 + ca
            off2 = off1 + ca
            g_issue(off1, ca, s1, d1, gs1, gd1)
            g_wait(off0, ca, s0, d0, gs0, gd0)

            @pl.when(p > 0)
            def _w0():
                st_wait(off0, ca, o0, st0)

            do_add(ca, s0, d0, o0)
            st_issue(off0, ca, o0, st0)

            @pl.when(off2 < nfull * ca)
            def _nx():
                g_issue(off2, ca, s0, d0, gs0, gd0)

            g_wait(off1, ca, s1, d1, gs1, gd1)

            @pl.when(p > 0)
            def _w1():
                st_wait(off1, ca, o1, st1)

            do_add(ca, s1, d1, o1)
            st_issue(off1, ca, o1, st1)
            return 0

        lax.fori_loop(0, npairs, pair, 0)
        assert odd == 0, "pair loop expects an even number of full chunks"
        st_wait(0, ca, o1, st1)
        if tail > 0:
            toff = nfull * ca
            sbt, dbt = s0.at[pl.ds(0, tail)], d0.at[pl.ds(0, tail)]
            obt = o0.at[pl.ds(0, tail)]
            g_issue(toff, tail, sbt, dbt, gs0, gd0)
            st_wait(0, ca, o0, st0)
            g_wait(toff, tail, sbt, dbt, gs0, gd0)
            do_add(tail, s0, d0, o0)
            pltpu.make_async_copy(
                obt, t_hbm.at[pl.ds(base + toff, tail)], st0).start()
            pltpu.make_async_copy(
                obt, t_hbm.at[pl.ds(base + toff, tail)], st0).wait()
        else:
            st_wait(0, ca, o0, st0)

    return pa(src_tab, dst_tab, src_idx, dst_idx)


def _edge_body(t_ref, e_ref, we_ref, b_ref, o_ref):
    w32 = t_ref[...]
    q = jnp.dot(e_ref[...], we_ref[...], preferred_element_type=jnp.float32)
    tsv = jax.lax.bitcast_convert_type(
        jax.lax.shift_left(w32, 16), jnp.float32)
    tgv = jax.lax.bitcast_convert_type(
        jax.lax.bitwise_and(w32, jnp.int32(-65536)), jnp.float32)
    ts = tsv + q[:, :256] + b_ref[0:1, :256]
    tg = tgv + q[:, 256:] + b_ref[0:1, 256:]
    sig = 1.0 / (1.0 + jnp.exp(-ts))
    sp = jnp.maximum(tg, 0.0) + jnp.log1p(jnp.exp(-jnp.abs(tg)))
    o_ref[...] = sig * sp


def _edge_mlp(t, edges_bf, we_bf, bias2d):
    e = t.shape[0]
    be = 640
    return pl.pallas_call(
        _edge_body,
        grid=(e // be,),
        in_specs=[
            pl.BlockSpec((be, 256), lambda i: (i, 0)),
            pl.BlockSpec((be, edges_bf.shape[1]), lambda i: (i, 0)),
            pl.BlockSpec(we_bf.shape, lambda i: (0, 0)),
            pl.BlockSpec(bias2d.shape, lambda i: (0, 0)),
        ],
        out_specs=pl.BlockSpec((be, 256), lambda i: (i, 0)),
        out_shape=jax.ShapeDtypeStruct((e, 256), jnp.float32),
    )(t, edges_bf, we_bf, bias2d)


ROWS_PER_TILE = 320
PC_CHUNK = 80


def _segment_add(atom, t, src_idx, bounds):
    e = t.shape[0]
    n = atom.shape[0]
    cc = PC_CHUNK
    rpt = ROWS_PER_TILE
    lastw = NW - 1
    lrows = n - lastw * rpt
    assert 0 < lrows <= rpt
    mesh = plsc.VectorSubcoreMesh(core_axis_name="c", subcore_axis_name="s")

    @functools.partial(
        pl.kernel,
        mesh=mesh,
        out_type=jax.ShapeDtypeStruct((n, 256), jnp.float32),
        scratch_types=[
            pltpu.VMEM((rpt, 256), jnp.float32),
            pltpu.VMEM((cc, 256), jnp.float32),
            pltpu.VMEM((cc, 256), jnp.float32),
            pltpu.VMEM((cc,), jnp.int32),
            pltpu.VMEM((cc,), jnp.int32),
            pltpu.VMEM((48,), jnp.int32),
            pltpu.SemaphoreType.DMA,
            pltpu.SemaphoreType.DMA,
        ],
        compiler_params=_SC_PARAMS,
    )
    def pc(atom_hbm, t_hbm, sidx_hbm, bnd_hbm, out_hbm, slab, tb0, tb1,
           ib0, ib1, bnd_v, sm0, sm1):
        w = lax.axis_index("s") * NC + lax.axis_index("c")
        nbase = w * rpt
        @pl.when(w < lastw)
        def _init_full():
            pltpu.sync_copy(atom_hbm.at[pl.ds(nbase, rpt)], slab)

        @pl.when(w == lastw)
        def _init_last():
            pltpu.sync_copy(atom_hbm.at[pl.ds(nbase, lrows)],
                            slab.at[pl.ds(0, lrows)])

        pltpu.sync_copy(bnd_hbm, bnd_v)

        iota = lax.iota(jnp.int32, LANES)
        wp = w + 1
        va = bnd_v[pl.ds((w // LANES) * LANES, LANES)]
        e_lo = jnp.sum(jnp.where(iota == w % LANES, va, 0))
        vb = bnd_v[pl.ds((wp // LANES) * LANES, LANES)]
        e_hi = jnp.sum(jnp.where(iota == wp % LANES, vb, 0))
        astart = (e_lo // 8) * 8
        nch = (e_hi - astart + cc - 1) // cc

        coloffs = [jnp.bitwise_and(iota + k, LANES - 1) for k in range(LANES)]
        erow = [g * LANES + iota for g in range(cc // LANES)]

        def ch_ero(ch):
            return jnp.minimum(astart + ch * cc, e - cc)

        def issue(ch, tb, ib, sm):
            ero = ch_ero(ch)
            pltpu.make_async_copy(t_hbm.at[pl.ds(ero, cc)], tb, sm).start()
            pltpu.make_async_copy(sidx_hbm.at[pl.ds(ero, cc)], ib, sm).start()

        def wait(ch, tb, ib, sm):
            ero = ch_ero(ch)
            pltpu.make_async_copy(t_hbm.at[pl.ds(ero, cc)], tb, sm).wait()
            pltpu.make_async_copy(sidx_hbm.at[pl.ds(ero, cc)], ib, sm).wait()

        def process(ch, tb, ib):
            eoff = astart + ch * cc
            ero = ch_ero(ch)
            cur_lo = jnp.maximum(e_lo, eoff)
            for g in range(cc // LANES):
                v = ib[pl.ds(g * LANES, LANES)]
                lv = jnp.clip(v - nbase, 0, rpt - 1)
                pos = ero + g * LANES + iota
                msk = (pos >= cur_lo) & (pos < e_hi)
                ev = erow[g]

                @pl.loop(0, 256 // LANES)
                def _cols(cg):
                    c0 = cg * LANES
                    for k in range(LANES):
                        cv = c0 + coloffs[k]
                        vals = plsc.load_gather(tb, [ev, cv])
                        plsc.addupdate_scatter(slab, [lv, cv], vals,
                                               mask=msk)

        @pl.when(nch > 0)
        def _pro():
            issue(0, tb0, ib0, sm0)

        def pair(p, _):
            ch0 = 2 * p
            ch1 = ch0 + 1

            @pl.when(ch1 < nch)
            def _i1():
                issue(ch1, tb1, ib1, sm1)

            wait(ch0, tb0, ib0, sm0)
            process(ch0, tb0, ib0)

            @pl.when(ch1 < nch)
            def _s1():
                @pl.when(ch1 + 1 < nch)
                def _i2():
                    issue(ch1 + 1, tb0, ib0, sm0)

                wait(ch1, tb1, ib1, sm1)
                process(ch1, tb1, ib1)

            return 0

        lax.fori_loop(0, (nch + 1) // 2, pair, 0)

        @pl.when(w < lastw)
        def _out_full():
            pltpu.sync_copy(slab, out_hbm.at[pl.ds(nbase, rpt)])

        @pl.when(w == lastw)
        def _out_last():
            pltpu.sync_copy(slab.at[pl.ds(0, lrows)],
                            out_hbm.at[pl.ds(nbase, lrows)])

    return pc(atom, t, src_idx, bounds)


def _softplus_body(x_ref, o_ref):
    x = x_ref[...]
    o_ref[...] = jnp.maximum(x, 0.0) + jnp.log1p(jnp.exp(-jnp.abs(x)))


def _softplus(x):
    n = x.shape[0]
    bm = 2000
    return pl.pallas_call(
        _softplus_body,
        grid=(n // bm,),
        in_specs=[pl.BlockSpec((bm, x.shape[1]), lambda i: (i, 0))],
        out_specs=pl.BlockSpec((bm, x.shape[1]), lambda i: (i, 0)),
        out_shape=jax.ShapeDtypeStruct(x.shape, jnp.float32),
    )(x)


def kernel(atom_features, edges_features, pair_indices, kernel_s, bias_s,
           kernel_g, bias_g):
    d = atom_features.shape[1]
    src = pair_indices[:, 0]
    dst = pair_indices[:, 1]

    w_all = jnp.concatenate(
        [kernel_s[:d], kernel_g[:d], kernel_s[d:2 * d], kernel_g[d:2 * d]],
        axis=1).astype(jnp.bfloat16)
    we = jnp.concatenate([kernel_s[2 * d:], kernel_g[2 * d:]],
                         axis=1).astype(jnp.bfloat16)
    bias2d = jnp.tile(jnp.concatenate([bias_s, bias_g])[None, :], (8, 1))

    n = atom_features.shape[0]
    bounds = jnp.pad(
        jnp.searchsorted(
            src, jnp.arange(NW + 1, dtype=jnp.int32) * ROWS_PER_TILE,
            side="left").astype(jnp.int32),
        (0, 48 - (NW + 1)), constant_values=src.shape[0])

    src_tab32, dst_tab32 = _make_tables(atom_features, w_all)
    t = _gather_sum(src_tab32, dst_tab32, src, dst)
    tact = _edge_mlp(t, edges_features.astype(jnp.bfloat16), we, bias2d)
    agg = _segment_add(atom_features, tact, src, bounds)
    return _softplus(agg)


# speedup vs baseline: 4.4251x; 1.2827x over previous
import dataclasses
import functools

import jax
import jax.numpy as jnp
from jax import lax
from jax.experimental import pallas as pl
from jax.experimental.pallas import tpu as pltpu
from jax.experimental.pallas import tpu_sc as plsc

NC = 2
NS = 16
NW = NC * NS
LANES = 16

_SC_PARAMS = pltpu.CompilerParams()
if "needs_layout_passes" in pltpu.CompilerParams.__dataclass_fields__:
    _SC_PARAMS = dataclasses.replace(_SC_PARAMS, needs_layout_passes=False)


def _pack_bf16_pair(s, g):
    u = jax.lax.bitcast_convert_type(s, jnp.int32)
    v = jax.lax.bitcast_convert_type(g, jnp.int32)
    one = jnp.int32(1)
    u = u + 0x7FFF + jax.lax.bitwise_and(jax.lax.shift_right_logical(u, 16),
                                         one)
    v = v + 0x7FFF + jax.lax.bitwise_and(jax.lax.shift_right_logical(v, 16),
                                         one)
    lo = jax.lax.shift_right_logical(u, 16)
    hi = jax.lax.bitwise_and(v, jnp.int32(-65536))
    return jax.lax.bitwise_or(lo, hi)


def _tables_body(x_ref, w_ref, src_ref, dst_ref):
    x = x_ref[...].astype(jnp.bfloat16)
    y = jnp.dot(x, w_ref[...], preferred_element_type=jnp.float32)
    src_ref[...] = _pack_bf16_pair(y[:, :256], y[:, 256:512])
    dst_ref[...] = _pack_bf16_pair(y[:, 512:768], y[:, 768:])


def _make_tables(atom, w_all):
    n = atom.shape[0]
    bm = 1000
    return pl.pallas_call(
        _tables_body,
        grid=(n // bm,),
        in_specs=[
            pl.BlockSpec((bm, atom.shape[1]), lambda i: (i, 0)),
            pl.BlockSpec(w_all.shape, lambda i: (0, 0)),
        ],
        out_specs=[
            pl.BlockSpec((bm, 256), lambda i: (i, 0)),
            pl.BlockSpec((bm, 256), lambda i: (i, 0)),
        ],
        out_shape=[
            jax.ShapeDtypeStruct((n, 256), jnp.int32),
            jax.ShapeDtypeStruct((n, 256), jnp.int32),
        ],
    )(atom, w_all)


def _gather_sum(src_tab, dst_tab, src_idx, dst_idx):
    e = src_idx.shape[0]
    ew = e // NW
    ca = 64
    nfull = ew // ca
    tail = ew - nfull * ca
    npairs = nfull // 2
    odd = nfull - npairs * 2
    mesh = plsc.VectorSubcoreMesh(core_axis_name="c", subcore_axis_name="s")

    @functools.partial(
        pl.kernel,
        mesh=mesh,
        out_type=jax.ShapeDtypeStruct((e, 256), jnp.int32),
        scratch_types=[
            pltpu.VMEM((ew,), jnp.int32),
            pltpu.VMEM((ew,), jnp.int32),
            pltpu.VMEM((ca, 256), jnp.int32),
            pltpu.VMEM((ca, 256), jnp.int32),
            pltpu.VMEM((ca, 256), jnp.int32),
            pltpu.VMEM((ca, 256), jnp.int32),
            pltpu.VMEM((ca, 256), jnp.int32),
            pltpu.VMEM((ca, 256), jnp.int32),
            pltpu.SemaphoreType.DMA,
            pltpu.SemaphoreType.DMA,
            pltpu.SemaphoreType.DMA,
            pltpu.SemaphoreType.DMA,
            pltpu.SemaphoreType.DMA,
            pltpu.SemaphoreType.DMA,
        ],
        compiler_params=_SC_PARAMS,
    )
    def pa(src_tab_hbm, dst_tab_hbm, sidx_hbm, didx_hbm, t_hbm,
           sidx_v, didx_v, s0, d0, s1, d1, o0, o1,
           gs0, gd0, gs1, gd1, st0, st1):
        wid = lax.axis_index("s") * NC + lax.axis_index("c")
        base = wid * ew
        pltpu.sync_copy(sidx_hbm.at[pl.ds(base, ew)], sidx_v)
        pltpu.sync_copy(didx_hbm.at[pl.ds(base, ew)], didx_v)

        def g_issue(off, nr, sb, db, ss, sd):
            pltpu.make_async_copy(
                src_tab_hbm.at[sidx_v.at[pl.ds(off, nr)]], sb, ss).start()
            pltpu.make_async_copy(
                dst_tab_hbm.at[didx_v.at[pl.ds(off, nr)]], db, sd).start()

        def g_wait(off, nr, sb, db, ss, sd):
            pltpu.make_async_copy(
                src_tab_hbm.at[sidx_v.at[pl.ds(off, nr)]], sb, ss).wait()
            pltpu.make_async_copy(
                dst_tab_hbm.at[didx_v.at[pl.ds(off, nr)]], db, sd).wait()

        def do_add(nr, sb, db, ob):
            @pl.loop(0, nr)
            def _row(r):
                for j in range(256 // LANES):
                    sl = (r, pl.ds(j * LANES, LANES))
                    a = plsc.bitcast(sb[sl], jnp.bfloat16)
                    b = plsc.bitcast(db[sl], jnp.bfloat16)
                    ob[sl] = plsc.bitcast(a + b, jnp.int32)

        def st_issue(off, nr, ob, sem):
            pltpu.make_async_copy(
                ob, t_hbm.at[pl.ds(base + off, nr)], sem).start()

        def st_wait(off, nr, ob, sem):
            pltpu.make_async_copy(
                ob, t_hbm.at[pl.ds(base + off, nr)], sem).wait()

        g_issue(0, ca, s0, d0, gs0, gd0)

        def pair(p, _):
            off0 = 2 * p * ca
            off1 = off0 + ca
            off2 = off1 + ca
            g_issue(off1, ca, s1, d1, gs1, gd1)
            g_wait(off0, ca, s0, d0, gs0, gd0)

            @pl.when(p > 0)
            def _w0():
                st_wait(off0, ca, o0, st0)

            do_add(ca, s0, d0, o0)
            st_issue(off0, ca, o0, st0)

            @pl.when(off2 < nfull * ca)
            def _nx():
                g_issue(off2, ca, s0, d0, gs0, gd0)

            g_wait(off1, ca, s1, d1, gs1, gd1)

            @pl.when(p > 0)
            def _w1():
                st_wait(off1, ca, o1, st1)

            do_add(ca, s1, d1, o1)
            st_issue(off1, ca, o1, st1)
            return 0

        lax.fori_loop(0, npairs, pair, 0)
        assert odd == 0, "pair loop expects an even number of full chunks"
        st_wait(0, ca, o1, st1)
        if tail > 0:
            toff = nfull * ca
            sbt, dbt = s0.at[pl.ds(0, tail)], d0.at[pl.ds(0, tail)]
            obt = o0.at[pl.ds(0, tail)]
            g_issue(toff, tail, sbt, dbt, gs0, gd0)
            st_wait(0, ca, o0, st0)
            g_wait(toff, tail, sbt, dbt, gs0, gd0)
            do_add(tail, s0, d0, o0)
            pltpu.make_async_copy(
                obt, t_hbm.at[pl.ds(base + toff, tail)], st0).start()
            pltpu.make_async_copy(
                obt, t_hbm.at[pl.ds(base + toff, tail)], st0).wait()
        else:
            st_wait(0, ca, o0, st0)

    return pa(src_tab, dst_tab, src_idx, dst_idx)


def _edge_body(t_ref, e_ref, we_ref, b_ref, o_ref):
    w32 = t_ref[...]
    q = jnp.dot(e_ref[...], we_ref[...], preferred_element_type=jnp.float32)
    tsv = jax.lax.bitcast_convert_type(
        jax.lax.shift_left(w32, 16), jnp.float32)
    tgv = jax.lax.bitcast_convert_type(
        jax.lax.bitwise_and(w32, jnp.int32(-65536)), jnp.float32)
    ts = tsv + q[:, :256] + b_ref[0:1, :256]
    tg = tgv + q[:, 256:] + b_ref[0:1, 256:]
    sig = 1.0 / (1.0 + jnp.exp(-ts))
    sp = jnp.maximum(tg, 0.0) + jnp.log1p(jnp.exp(-jnp.abs(tg)))
    o_ref[...] = sig * sp


def _edge_mlp(t, edges_bf, we_bf, bias2d):
    e = t.shape[0]
    be = 640
    return pl.pallas_call(
        _edge_body,
        grid=(e // be,),
        in_specs=[
            pl.BlockSpec((be, 256), lambda i: (i, 0)),
            pl.BlockSpec((be, edges_bf.shape[1]), lambda i: (i, 0)),
            pl.BlockSpec(we_bf.shape, lambda i: (0, 0)),
            pl.BlockSpec(bias2d.shape, lambda i: (0, 0)),
        ],
        out_specs=pl.BlockSpec((be, 256), lambda i: (i, 0)),
        out_shape=jax.ShapeDtypeStruct((e, 256), jnp.float32),
    )(t, edges_bf, we_bf, bias2d)


ROWS_PER_TILE = 320
PC_CHUNK = 80


def _segment_add(atom, t, src_idx, bounds):
    e = t.shape[0]
    n = atom.shape[0]
    cc = PC_CHUNK
    rpt = ROWS_PER_TILE
    lastw = NW - 1
    lrows = n - lastw * rpt
    assert 0 < lrows <= rpt
    mesh = plsc.VectorSubcoreMesh(core_axis_name="c", subcore_axis_name="s")

    @functools.partial(
        pl.kernel,
        mesh=mesh,
        out_type=jax.ShapeDtypeStruct((n, 256), jnp.float32),
        scratch_types=[
            pltpu.VMEM((rpt, 256), jnp.float32),
            pltpu.VMEM((cc, 256), jnp.float32),
            pltpu.VMEM((cc, 256), jnp.float32),
            pltpu.VMEM((cc,), jnp.int32),
            pltpu.VMEM((cc,), jnp.int32),
            pltpu.VMEM((48,), jnp.int32),
            pltpu.SemaphoreType.DMA,
            pltpu.SemaphoreType.DMA,
        ],
        compiler_params=_SC_PARAMS,
    )
    def pc(atom_hbm, t_hbm, sidx_hbm, bnd_hbm, out_hbm, slab, tb0, tb1,
           ib0, ib1, bnd_v, sm0, sm1):
        w = lax.axis_index("s") * NC + lax.axis_index("c")
        nbase = w * rpt
        @pl.when(w < lastw)
        def _init_full():
            pltpu.sync_copy(atom_hbm.at[pl.ds(nbase, rpt)], slab)

        @pl.when(w == lastw)
        def _init_last():
            pltpu.sync_copy(atom_hbm.at[pl.ds(nbase, lrows)],
                            slab.at[pl.ds(0, lrows)])

        pltpu.sync_copy(bnd_hbm, bnd_v)

        iota = lax.iota(jnp.int32, LANES)
        wp = w + 1
        va = bnd_v[pl.ds((w // LANES) * LANES, LANES)]
        e_lo = jnp.sum(jnp.where(iota == w % LANES, va, 0))
        vb = bnd_v[pl.ds((wp // LANES) * LANES, LANES)]
        e_hi = jnp.sum(jnp.where(iota == wp % LANES, vb, 0))
        astart = (e_lo // 8) * 8
        nch = (e_hi - astart + cc - 1) // cc

        coloffs = [jnp.bitwise_and(iota + k, LANES - 1) for k in range(LANES)]
        erow = [g * LANES + iota for g in range(cc // LANES)]

        def ch_ero(ch):
            return jnp.minimum(astart + ch * cc, e - cc)

        def issue(ch, tb, ib, sm):
            ero = ch_ero(ch)
            pltpu.make_async_copy(t_hbm.at[pl.ds(ero, cc)], tb, sm).start()
            pltpu.make_async_copy(sidx_hbm.at[pl.ds(ero, cc)], ib, sm).start()

        def wait(ch, tb, ib, sm):
            ero = ch_ero(ch)
            pltpu.make_async_copy(t_hbm.at[pl.ds(ero, cc)], tb, sm).wait()
            pltpu.make_async_copy(sidx_hbm.at[pl.ds(ero, cc)], ib, sm).wait()

        def process(ch, tb, ib):
            eoff = astart + ch * cc
            ero = ch_ero(ch)
            cur_lo = jnp.maximum(e_lo, eoff)
            for g in range(cc // LANES):
                v = ib[pl.ds(g * LANES, LANES)]
                lv = jnp.clip(v - nbase, 0, rpt - 1)
                pos = ero + g * LANES + iota
                msk = (pos >= cur_lo) & (pos < e_hi)
                ev = erow[g]

                @pl.loop(0, 256 // LANES)
                def _cols(cg):
                    c0 = cg * LANES
                    cvs = [c0 + coloffs[k] for k in range(LANES)]
                    vals = [plsc.load_gather(tb, [ev, cv]) for cv in cvs]
                    for cv, val in zip(cvs, vals):
                        plsc.addupdate_scatter(slab, [lv, cv], val, mask=msk)

        @pl.when(nch > 0)
        def _pro():
            issue(0, tb0, ib0, sm0)

        def pair(p, _):
            ch0 = 2 * p
            ch1 = ch0 + 1

            @pl.when(ch1 < nch)
            def _i1():
                issue(ch1, tb1, ib1, sm1)

            wait(ch0, tb0, ib0, sm0)
            process(ch0, tb0, ib0)

            @pl.when(ch1 < nch)
            def _s1():
                @pl.when(ch1 + 1 < nch)
                def _i2():
                    issue(ch1 + 1, tb0, ib0, sm0)

                wait(ch1, tb1, ib1, sm1)
                process(ch1, tb1, ib1)

            return 0

        lax.fori_loop(0, (nch + 1) // 2, pair, 0)

        @pl.when(w < lastw)
        def _out_full():
            pltpu.sync_copy(slab, out_hbm.at[pl.ds(nbase, rpt)])

        @pl.when(w == lastw)
        def _out_last():
            pltpu.sync_copy(slab.at[pl.ds(0, lrows)],
                            out_hbm.at[pl.ds(nbase, lrows)])

    return pc(atom, t, src_idx, bounds)


def _softplus_body(x_ref, o_ref):
    x = x_ref[...]
    o_ref[...] = jnp.maximum(x, 0.0) + jnp.log1p(jnp.exp(-jnp.abs(x)))


def _softplus(x):
    n = x.shape[0]
    bm = 2000
    return pl.pallas_call(
        _softplus_body,
        grid=(n // bm,),
        in_specs=[pl.BlockSpec((bm, x.shape[1]), lambda i: (i, 0))],
        out_specs=pl.BlockSpec((bm, x.shape[1]), lambda i: (i, 0)),
        out_shape=jax.ShapeDtypeStruct(x.shape, jnp.float32),
    )(x)


def kernel(atom_features, edges_features, pair_indices, kernel_s, bias_s,
           kernel_g, bias_g):
    d = atom_features.shape[1]
    src = pair_indices[:, 0]
    dst = pair_indices[:, 1]

    w_all = jnp.concatenate(
        [kernel_s[:d], kernel_g[:d], kernel_s[d:2 * d], kernel_g[d:2 * d]],
        axis=1).astype(jnp.bfloat16)
    we = jnp.concatenate([kernel_s[2 * d:], kernel_g[2 * d:]],
                         axis=1).astype(jnp.bfloat16)
    bias2d = jnp.tile(jnp.concatenate([bias_s, bias_g])[None, :], (8, 1))

    n = atom_features.shape[0]
    bounds = jnp.pad(
        jnp.searchsorted(
            src, jnp.arange(NW + 1, dtype=jnp.int32) * ROWS_PER_TILE,
            side="left").astype(jnp.int32),
        (0, 48 - (NW + 1)), constant_values=src.shape[0])

    src_tab32, dst_tab32 = _make_tables(atom_features, w_all)
    t = _gather_sum(src_tab32, dst_tab32, src, dst)
    tact = _edge_mlp(t, edges_features.astype(jnp.bfloat16), we, bias2d)
    agg = _segment_add(atom_features, tact, src, bounds)
    return _softplus(agg)


# speedup vs baseline: 4.5280x; 1.0232x over previous
import dataclasses
import functools

import jax
import jax.numpy as jnp
from jax import lax
from jax.experimental import pallas as pl
from jax.experimental.pallas import tpu as pltpu
from jax.experimental.pallas import tpu_sc as plsc

NC = 2
NS = 16
NW = NC * NS
LANES = 16

_SC_PARAMS = pltpu.CompilerParams()
if "needs_layout_passes" in pltpu.CompilerParams.__dataclass_fields__:
    _SC_PARAMS = dataclasses.replace(_SC_PARAMS, needs_layout_passes=False)


def _pack_bf16_pair(s, g):
    u = jax.lax.bitcast_convert_type(s, jnp.int32)
    v = jax.lax.bitcast_convert_type(g, jnp.int32)
    one = jnp.int32(1)
    u = u + 0x7FFF + jax.lax.bitwise_and(jax.lax.shift_right_logical(u, 16),
                                         one)
    v = v + 0x7FFF + jax.lax.bitwise_and(jax.lax.shift_right_logical(v, 16),
                                         one)
    lo = jax.lax.shift_right_logical(u, 16)
    hi = jax.lax.bitwise_and(v, jnp.int32(-65536))
    return jax.lax.bitwise_or(lo, hi)


def _tables_body(x_ref, w_ref, src_ref, dst_ref):
    x = x_ref[...].astype(jnp.bfloat16)
    y = jnp.dot(x, w_ref[...], preferred_element_type=jnp.float32)
    src_ref[...] = _pack_bf16_pair(y[:, :256], y[:, 256:512])
    dst_ref[...] = _pack_bf16_pair(y[:, 512:768], y[:, 768:])


def _make_tables(atom, w_all):
    n = atom.shape[0]
    bm = 1000
    return pl.pallas_call(
        _tables_body,
        grid=(n // bm,),
        in_specs=[
            pl.BlockSpec((bm, atom.shape[1]), lambda i: (i, 0)),
            pl.BlockSpec(w_all.shape, lambda i: (0, 0)),
        ],
        out_specs=[
            pl.BlockSpec((bm, 256), lambda i: (i, 0)),
            pl.BlockSpec((bm, 256), lambda i: (i, 0)),
        ],
        out_shape=[
            jax.ShapeDtypeStruct((n, 256), jnp.int32),
            jax.ShapeDtypeStruct((n, 256), jnp.int32),
        ],
    )(atom, w_all)


def _gather_sum(src_tab, dst_tab, src_idx, dst_idx):
    e = src_idx.shape[0]
    ew = e // NW
    ca = 64
    nfull = ew // ca
    tail = ew - nfull * ca
    npairs = nfull // 2
    odd = nfull - npairs * 2
    mesh = plsc.VectorSubcoreMesh(core_axis_name="c", subcore_axis_name="s")

    @functools.partial(
        pl.kernel,
        mesh=mesh,
        out_type=jax.ShapeDtypeStruct((e, 256), jnp.int32),
        scratch_types=[
            pltpu.VMEM((ew,), jnp.int32),
            pltpu.VMEM((ew,), jnp.int32),
            pltpu.VMEM((ca, 256), jnp.int32),
            pltpu.VMEM((ca, 256), jnp.int32),
            pltpu.VMEM((ca, 256), jnp.int32),
            pltpu.VMEM((ca, 256), jnp.int32),
            pltpu.VMEM((ca, 256), jnp.int32),
            pltpu.VMEM((ca, 256), jnp.int32),
            pltpu.SemaphoreType.DMA,
            pltpu.SemaphoreType.DMA,
            pltpu.SemaphoreType.DMA,
            pltpu.SemaphoreType.DMA,
            pltpu.SemaphoreType.DMA,
            pltpu.SemaphoreType.DMA,
        ],
        compiler_params=_SC_PARAMS,
    )
    def pa(src_tab_hbm, dst_tab_hbm, sidx_hbm, didx_hbm, t_hbm,
           sidx_v, didx_v, s0, d0, s1, d1, o0, o1,
           gs0, gd0, gs1, gd1, st0, st1):
        wid = lax.axis_index("s") * NC + lax.axis_index("c")
        base = wid * ew
        pltpu.sync_copy(sidx_hbm.at[pl.ds(base, ew)], sidx_v)
        pltpu.sync_copy(didx_hbm.at[pl.ds(base, ew)], didx_v)

        def g_issue(off, nr, sb, db, ss, sd):
            pltpu.make_async_copy(
                src_tab_hbm.at[sidx_v.at[pl.ds(off, nr)]], sb, ss).start()
            pltpu.make_async_copy(
                dst_tab_hbm.at[didx_v.at[pl.ds(off, nr)]], db, sd).start()

        def g_wait(off, nr, sb, db, ss, sd):
            pltpu.make_async_copy(
                src_tab_hbm.at[sidx_v.at[pl.ds(off, nr)]], sb, ss).wait()
            pltpu.make_async_copy(
                dst_tab_hbm.at[didx_v.at[pl.ds(off, nr)]], db, sd).wait()

        def do_add(nr, sb, db, ob):
            @pl.loop(0, nr)
            def _row(r):
                for j in range(256 // LANES):
                    sl = (r, pl.ds(j * LANES, LANES))
                    a = plsc.bitcast(sb[sl], jnp.bfloat16)
                    b = plsc.bitcast(db[sl], jnp.bfloat16)
                    ob[sl] = plsc.bitcast(a + b, jnp.int32)

        def st_issue(off, nr, ob, sem):
            pltpu.make_async_copy(
                ob, t_hbm.at[pl.ds(base + off, nr)], sem).start()

        def st_wait(off, nr, ob, sem):
            pltpu.make_async_copy(
                ob, t_hbm.at[pl.ds(base + off, nr)], sem).wait()

        g_issue(0, ca, s0, d0, gs0, gd0)

        def pair(p, _):
            off0 = 2 * p * ca
            off1 = off0 + ca
            off2 = off1 + ca
            g_issue(off1, ca, s1, d1, gs1, gd1)
            g_wait(off0, ca, s0, d0, gs0, gd0)

            @pl.when(p > 0)
            def _w0():
                st_wait(off0, ca, o0, st0)

            do_add(ca, s0, d0, o0)
            st_issue(off0, ca, o0, st0)

            @pl.when(off2 < nfull * ca)
            def _nx():
                g_issue(off2, ca, s0, d0, gs0, gd0)

            g_wait(off1, ca, s1, d1, gs1, gd1)

            @pl.when(p > 0)
            def _w1():
                st_wait(off1, ca, o1, st1)

            do_add(ca, s1, d1, o1)
            st_issue(off1, ca, o1, st1)
            return 0

        lax.fori_loop(0, npairs, pair, 0)
        assert odd == 0, "pair loop expects an even number of full chunks"
        st_wait(0, ca, o1, st1)
        if tail > 0:
            toff = nfull * ca
            sbt, dbt = s0.at[pl.ds(0, tail)], d0.at[pl.ds(0, tail)]
            obt = o0.at[pl.ds(0, tail)]
            g_issue(toff, tail, sbt, dbt, gs0, gd0)
            st_wait(0, ca, o0, st0)
            g_wait(toff, tail, sbt, dbt, gs0, gd0)
            do_add(tail, s0, d0, o0)
            pltpu.make_async_copy(
                obt, t_hbm.at[pl.ds(base + toff, tail)], st0).start()
            pltpu.make_async_copy(
                obt, t_hbm.at[pl.ds(base + toff, tail)], st0).wait()
        else:
            st_wait(0, ca, o0, st0)

    return pa(src_tab, dst_tab, src_idx, dst_idx)


def _edge_body(t_ref, e_ref, we_ref, b_ref, o_ref):
    w32 = t_ref[...]
    q = jnp.dot(e_ref[...], we_ref[...], preferred_element_type=jnp.float32)
    tsv = jax.lax.bitcast_convert_type(
        jax.lax.shift_left(w32, 16), jnp.float32)
    tgv = jax.lax.bitcast_convert_type(
        jax.lax.bitwise_and(w32, jnp.int32(-65536)), jnp.float32)
    ts = tsv + q[:, :256] + b_ref[0:1, :256]
    tg = tgv + q[:, 256:] + b_ref[0:1, 256:]
    sig = 1.0 / (1.0 + jnp.exp(-ts))
    sp = jnp.maximum(tg, 0.0) + jnp.log1p(jnp.exp(-jnp.abs(tg)))
    t = sig * sp
    o_ref[...] = _pack_bf16_pair(t[:, :128], t[:, 128:])


def _edge_mlp(t, edges_bf, we_bf, bias2d):
    e = t.shape[0]
    be = 640
    return pl.pallas_call(
        _edge_body,
        grid=(e // be,),
        in_specs=[
            pl.BlockSpec((be, 256), lambda i: (i, 0)),
            pl.BlockSpec((be, edges_bf.shape[1]), lambda i: (i, 0)),
            pl.BlockSpec(we_bf.shape, lambda i: (0, 0)),
            pl.BlockSpec(bias2d.shape, lambda i: (0, 0)),
        ],
        out_specs=pl.BlockSpec((be, 128), lambda i: (i, 0)),
        out_shape=jax.ShapeDtypeStruct((e, 128), jnp.int32),
    )(t, edges_bf, we_bf, bias2d)


ROWS_PER_TILE = 320
PC_CHUNK = 80


def _segment_add(atom, t, src_idx, bounds):
    e = t.shape[0]
    n = atom.shape[0]
    cc = PC_CHUNK
    rpt = ROWS_PER_TILE
    lastw = NW - 1
    lrows = n - lastw * rpt
    assert 0 < lrows <= rpt
    mesh = plsc.VectorSubcoreMesh(core_axis_name="c", subcore_axis_name="s")

    @functools.partial(
        pl.kernel,
        mesh=mesh,
        out_type=jax.ShapeDtypeStruct((n, 256), jnp.float32),
        scratch_types=[
            pltpu.VMEM((rpt, 256), jnp.float32),
            pltpu.VMEM((cc, 128), jnp.int32),
            pltpu.VMEM((cc, 128), jnp.int32),
            pltpu.VMEM((cc,), jnp.int32),
            pltpu.VMEM((cc,), jnp.int32),
            pltpu.VMEM((48,), jnp.int32),
            pltpu.SemaphoreType.DMA,
            pltpu.SemaphoreType.DMA,
        ],
        compiler_params=_SC_PARAMS,
    )
    def pc(atom_hbm, t_hbm, sidx_hbm, bnd_hbm, out_hbm, slab, tb0, tb1,
           ib0, ib1, bnd_v, sm0, sm1):
        w = lax.axis_index("s") * NC + lax.axis_index("c")
        nbase = w * rpt
        @pl.when(w < lastw)
        def _init_full():
            pltpu.sync_copy(atom_hbm.at[pl.ds(nbase, rpt)], slab)

        @pl.when(w == lastw)
        def _init_last():
            pltpu.sync_copy(atom_hbm.at[pl.ds(nbase, lrows)],
                            slab.at[pl.ds(0, lrows)])

        pltpu.sync_copy(bnd_hbm, bnd_v)

        iota = lax.iota(jnp.int32, LANES)
        wp = w + 1
        va = bnd_v[pl.ds((w // LANES) * LANES, LANES)]
        e_lo = jnp.sum(jnp.where(iota == w % LANES, va, 0))
        vb = bnd_v[pl.ds((wp // LANES) * LANES, LANES)]
        e_hi = jnp.sum(jnp.where(iota == wp % LANES, vb, 0))
        astart = (e_lo // 8) * 8
        nch = (e_hi - astart + cc - 1) // cc

        coloffs = [jnp.bitwise_and(iota + k, LANES - 1) for k in range(LANES)]
        erow = [g * LANES + iota for g in range(cc // LANES)]

        def ch_ero(ch):
            return jnp.minimum(astart + ch * cc, e - cc)

        def issue(ch, tb, ib, sm):
            ero = ch_ero(ch)
            pltpu.make_async_copy(t_hbm.at[pl.ds(ero, cc)], tb, sm).start()
            pltpu.make_async_copy(sidx_hbm.at[pl.ds(ero, cc)], ib, sm).start()

        def wait(ch, tb, ib, sm):
            ero = ch_ero(ch)
            pltpu.make_async_copy(t_hbm.at[pl.ds(ero, cc)], tb, sm).wait()
            pltpu.make_async_copy(sidx_hbm.at[pl.ds(ero, cc)], ib, sm).wait()

        def process(ch, tb, ib):
            eoff = astart + ch * cc
            ero = ch_ero(ch)
            cur_lo = jnp.maximum(e_lo, eoff)
            for g in range(cc // LANES):
                v = ib[pl.ds(g * LANES, LANES)]
                lv = jnp.clip(v - nbase, 0, rpt - 1)
                pos = ero + g * LANES + iota
                msk = (pos >= cur_lo) & (pos < e_hi)
                ev = erow[g]

                @pl.loop(0, 128 // 8)
                def _cols(cg):
                    c0 = cg * 8
                    cvs = [jnp.bitwise_and(c0 + coloffs[k], 127)
                           for k in range(8)]
                    ws = [plsc.load_gather(tb, [ev, cv]) for cv in cvs]
                    los = [plsc.bitcast(jnp.left_shift(wv, 16), jnp.float32)
                           for wv in ws]
                    his = [plsc.bitcast(
                        jax.lax.bitwise_and(wv, jnp.int32(-65536)),
                        jnp.float32) for wv in ws]
                    for cv, lo in zip(cvs, los):
                        plsc.addupdate_scatter(slab, [lv, cv], lo, mask=msk)
                    for cv, hi in zip(cvs, his):
                        plsc.addupdate_scatter(slab, [lv, cv + 128], hi,
                                               mask=msk)

        @pl.when(nch > 0)
        def _pro():
            issue(0, tb0, ib0, sm0)

        def pair(p, _):
            ch0 = 2 * p
            ch1 = ch0 + 1

            @pl.when(ch1 < nch)
            def _i1():
                issue(ch1, tb1, ib1, sm1)

            wait(ch0, tb0, ib0, sm0)
            process(ch0, tb0, ib0)

            @pl.when(ch1 < nch)
            def _s1():
                @pl.when(ch1 + 1 < nch)
                def _i2():
                    issue(ch1 + 1, tb0, ib0, sm0)

                wait(ch1, tb1, ib1, sm1)
                process(ch1, tb1, ib1)

            return 0

        lax.fori_loop(0, (nch + 1) // 2, pair, 0)

        @pl.when(w < lastw)
        def _out_full():
            pltpu.sync_copy(slab, out_hbm.at[pl.ds(nbase, rpt)])

        @pl.when(w == lastw)
        def _out_last():
            pltpu.sync_copy(slab.at[pl.ds(0, lrows)],
                            out_hbm.at[pl.ds(nbase, lrows)])

    return pc(atom, t, src_idx, bounds)


def _softplus_body(x_ref, o_ref):
    x = x_ref[...]
    o_ref[...] = jnp.maximum(x, 0.0) + jnp.log1p(jnp.exp(-jnp.abs(x)))


def _softplus(x):
    n = x.shape[0]
    bm = 2000
    return pl.pallas_call(
        _softplus_body,
        grid=(n // bm,),
        in_specs=[pl.BlockSpec((bm, x.shape[1]), lambda i: (i, 0))],
        out_specs=pl.BlockSpec((bm, x.shape[1]), lambda i: (i, 0)),
        out_shape=jax.ShapeDtypeStruct(x.shape, jnp.float32),
    )(x)


def kernel(atom_features, edges_features, pair_indices, kernel_s, bias_s,
           kernel_g, bias_g):
    d = atom_features.shape[1]
    src = pair_indices[:, 0]
    dst = pair_indices[:, 1]

    w_all = jnp.concatenate(
        [kernel_s[:d], kernel_g[:d], kernel_s[d:2 * d], kernel_g[d:2 * d]],
        axis=1).astype(jnp.bfloat16)
    we = jnp.concatenate([kernel_s[2 * d:], kernel_g[2 * d:]],
                         axis=1).astype(jnp.bfloat16)
    bias2d = jnp.tile(jnp.concatenate([bias_s, bias_g])[None, :], (8, 1))

    n = atom_features.shape[0]
    bounds = jnp.pad(
        jnp.searchsorted(
            src, jnp.arange(NW + 1, dtype=jnp.int32) * ROWS_PER_TILE,
            side="left").astype(jnp.int32),
        (0, 48 - (NW + 1)), constant_values=src.shape[0])

    src_tab32, dst_tab32 = _make_tables(atom_features, w_all)
    t = _gather_sum(src_tab32, dst_tab32, src, dst)
    tact = _edge_mlp(t, edges_features.astype(jnp.bfloat16), we, bias2d)
    agg = _segment_add(atom_features, tact, src, bounds)
    return _softplus(agg)
